# Initial kernel scaffold; baseline (speedup 1.0000x reference)
import jax
import jax.numpy as jnp
from jax.experimental import pallas as pl


def kernel(nfeat, efeat, edge_index, eW1, eb1, eW2, eb2, eW3, eb3, eg, ebt, nW1, nb1, nW2, nb2, nW3, nb3, ng, nbt):
    raise NotImplementedError("write your pallas kernel here")


# speedup vs baseline: 3.0537x; 3.0537x over previous
import functools

import jax
import jax.numpy as jnp
from jax import lax
from jax.experimental import pallas as pl
from jax.experimental.pallas import tpu as pltpu
from jax.experimental.pallas import tpu_sc as plsc

N = 10000
E = 320000
D = 128
N_BLK = 2

NC, NS = 2, 16
NW = NC * NS
EW = E // NW
K = 80
CHUNKS = EW // K
NPAD = 10240
ROWS_PER_SUB = NPAD // NS

_mesh = plsc.VectorSubcoreMesh(core_axis_name="c", subcore_axis_name="s")


@functools.partial(
    pl.kernel,
    mesh=_mesh,
    out_type=jax.ShapeDtypeStruct((E, D), jnp.float32),
    scratch_types=[
        pltpu.VMEM((K,), jnp.int32),
        pltpu.VMEM((K,), jnp.int32),
        pltpu.VMEM((K, D), jnp.float32),
        pltpu.VMEM((K, D), jnp.float32),
        pltpu.SemaphoreType.DMA,
        pltpu.SemaphoreType.DMA,
    ],
)
def _sc_gather_add(p_hbm, q_hbm, src_hbm, dst_hbm, t_hbm,
                   idx_s, idx_d, rows_p, rows_q, sem_p, sem_q):
    wid = lax.axis_index("s") * NC + lax.axis_index("c")
    base = wid * EW

    @pl.loop(0, CHUNKS)
    def _chunk(j):
        off = base + j * K
        pltpu.sync_copy(src_hbm.at[pl.ds(off, K)], idx_s)
        pltpu.sync_copy(dst_hbm.at[pl.ds(off, K)], idx_d)
        cp_p = pltpu.async_copy(p_hbm.at[idx_s], rows_p, sem_p)
        cp_q = pltpu.async_copy(q_hbm.at[idx_d], rows_q, sem_q)
        cp_p.wait()
        cp_q.wait()

        @pl.loop(0, K)
        def _row(r):
            @pl.loop(0, D, step=16)
            def _col(c):
                slc = (pl.ds(r, 1), pl.ds(c, 16))
                rows_p.at[*slc][...] = rows_p.at[*slc][...] + rows_q.at[*slc][...]

        pltpu.sync_copy(rows_p, t_hbm.at[pl.ds(off, K)])


@functools.partial(
    pl.kernel,
    mesh=_mesh,
    out_type=jax.ShapeDtypeStruct((NC * NPAD, D), jnp.float32),
    scratch_types=[
        pltpu.VMEM((K,), jnp.int32),
        pltpu.VMEM((K, D), jnp.float32),
        pltpu.VMEM((K, D), jnp.float32),
        pltpu.VMEM_SHARED((NPAD, D), jnp.float32),
    ],
)
def _sc_scatter_add(e_hbm, dst_hbm, z_hbm, out_hbm, idx_v, rows_v, zb, acc):
    cid = lax.axis_index("c")
    sid = lax.axis_index("s")

    pltpu.sync_copy(z_hbm, zb)

    @pl.loop(0, ROWS_PER_SUB // K)
    def _z(t):
        pltpu.sync_copy(zb, acc.at[pl.ds(sid * ROWS_PER_SUB + t * K, K)])

    plsc.subcore_barrier()

    base = cid * (E // NC) + sid * EW

    @pl.loop(0, CHUNKS)
    def _chunk(j):
        off = base + j * K
        pltpu.sync_copy(dst_hbm.at[pl.ds(off, K)], idx_v)
        pltpu.sync_copy(e_hbm.at[pl.ds(off, K)], rows_v)
        pltpu.sync_copy(rows_v, acc.at[idx_v], add=True)

    plsc.subcore_barrier()
    pltpu.sync_copy(
        acc.at[pl.ds(sid * ROWS_PER_SUB, ROWS_PER_SUB)],
        out_hbm.at[pl.ds(cid * NPAD + sid * ROWS_PER_SUB, ROWS_PER_SUB)])


TE = 2560
TN = 2000


def _ln_res(h, x, g, bt):
    mu = jnp.mean(h, axis=-1, keepdims=True)
    d = h - mu
    var = jnp.mean(d * d, axis=-1, keepdims=True)
    return d * lax.rsqrt(var + 1e-5) * g + bt + x


def _proj_body(nf_ref, ws, wd, p_ref, q_ref):
    x = nf_ref[...]
    p_ref[...] = jnp.dot(x, ws[...], preferred_element_type=jnp.float32)
    q_ref[...] = jnp.dot(x, wd[...], preferred_element_type=jnp.float32)


def _proj(nf, ws, wd):
    row = pl.BlockSpec((TN, D), lambda i: (i, 0))
    full = pl.BlockSpec((D, D), lambda i: (0, 0))
    return pl.pallas_call(
        _proj_body,
        grid=(N // TN,),
        in_specs=[row, full, full],
        out_specs=[row, row],
        out_shape=[jax.ShapeDtypeStruct((N, D), jnp.float32),
                   jax.ShapeDtypeStruct((N, D), jnp.float32)],
    )(nf, ws, wd)


def _edge_body(ef_ref, t_ref, w1, w2, w3, b1, b2, b3, g, bt, out_ref):
    x = ef_ref[...]
    h = jnp.dot(x, w1[...], preferred_element_type=jnp.float32)
    h = jnp.maximum(h + t_ref[...] + b1[...], 0.0)
    h = jnp.maximum(jnp.dot(h, w2[...], preferred_element_type=jnp.float32) + b2[...], 0.0)
    h = jnp.dot(h, w3[...], preferred_element_type=jnp.float32) + b3[...]
    out_ref[...] = _ln_res(h, x, g[...], bt[...])


def _edge_mlp(ef, t, w1, w2, w3, b1, b2, b3, g, bt):
    row = pl.BlockSpec((TE, D), lambda i: (i, 0))
    full = pl.BlockSpec((D, D), lambda i: (0, 0))
    vec = pl.BlockSpec((1, D), lambda i: (0, 0))
    return pl.pallas_call(
        _edge_body,
        grid=(E // TE,),
        in_specs=[row, row, full, full, full, vec, vec, vec, vec, vec],
        out_specs=row,
        out_shape=jax.ShapeDtypeStruct((E, D), jnp.float32),
    )(ef, t, w1, w2, w3, b1, b2, b3, g, bt)


def _node_body(p0_ref, p1_ref, nf_ref, w1a, w1b, w2, w3, b1, b2, b3, g, bt,
               out_ref):
    x = nf_ref[...]
    agg = p0_ref[...] + p1_ref[...]
    h = (jnp.dot(agg, w1a[...], preferred_element_type=jnp.float32)
         + jnp.dot(x, w1b[...], preferred_element_type=jnp.float32))
    h = jnp.maximum(h + b1[...], 0.0)
    h = jnp.maximum(jnp.dot(h, w2[...], preferred_element_type=jnp.float32) + b2[...], 0.0)
    h = jnp.dot(h, w3[...], preferred_element_type=jnp.float32) + b3[...]
    out_ref[...] = _ln_res(h, x, g[...], bt[...])


def _node_mlp(p0, p1, nf, w1a, w1b, w2, w3, b1, b2, b3, g, bt):
    row = pl.BlockSpec((TN, D), lambda i: (i, 0))
    full = pl.BlockSpec((D, D), lambda i: (0, 0))
    vec = pl.BlockSpec((1, D), lambda i: (0, 0))
    return pl.pallas_call(
        _node_body,
        grid=(N // TN,),
        in_specs=[row, row, row, full, full, full, full,
                  vec, vec, vec, vec, vec],
        out_specs=row,
        out_shape=jax.ShapeDtypeStruct((N, D), jnp.float32),
    )(p0, p1, nf, w1a, w1b, w2, w3, b1, b2, b3, g, bt)


def kernel(nfeat, efeat, edge_index, eW1, eb1, eW2, eb2, eW3, eb3, eg, ebt,
           nW1, nb1, nW2, nb2, nW3, nb3, ng, nbt):
    src = edge_index[0].astype(jnp.int32)
    dst = edge_index[1].astype(jnp.int32)
    zeros = jnp.zeros((K, D), jnp.float32)

    for i in range(N_BLK):
        w1e = eW1[i, :D]
        w1s = eW1[i, D:2 * D]
        w1d = eW1[i, 2 * D:]
        p, q = _proj(nfeat, w1s, w1d)
        t = _sc_gather_add(p, q, src, dst)
        efeat = _edge_mlp(efeat, t, w1e, eW2[i], eW3[i],
                          eb1[i][None], eb2[i][None], eb3[i][None],
                          eg[i][None], ebt[i][None])
        parts = _sc_scatter_add(efeat, dst, zeros)
        nfeat = _node_mlp(parts[:N], parts[NPAD:NPAD + N], nfeat,
                          nW1[i, :D], nW1[i, D:], nW2[i], nW3[i],
                          nb1[i][None], nb2[i][None], nb3[i][None],
                          ng[i][None], nbt[i][None])
    return nfeat
